# baseline (speedup 1.0000x reference)
import functools

import jax
import jax.numpy as jnp
from jax import lax
from jax.experimental import pallas as pl
from jax.experimental.pallas import tpu as pltpu
from jax.experimental.pallas import tpu_sc as plsc

N_CLS = 10000
N_CTX = 16
SUF = 60
SEQ = 77
D = 128
B = 1024
NV = 4

NW = 32
PER_TILE = NV * B // NW
C = 8
NCHUNK = PER_TILE // C
TILES_PER_V = NW // NV
IDX_ROWS = PER_TILE // C


def _sc_body(cls2d, pn, cn, sn, pp, cp, sp, pe, ce, se, ps, cs, ss,
             out, idx_v, bbuf, gs0, gs1, ws0, ws1):
  cid = lax.axis_index("c")
  sid = lax.axis_index("s")
  wid = sid * 2 + cid
  v = wid // TILES_PER_V
  r0 = (wid % TILES_PER_V) * IDX_ROWS
  i0_tile = wid * PER_TILE

  pltpu.sync_copy(cls2d.at[pl.ds(r0, IDX_ROWS)], idx_v)

  gsems = (gs0, gs1)
  wsems = (ws0, ws1)

  def run(pref, ctxt, suft):
    def start_gather(g, t):
      idx = idx_v.at[g]
      pltpu.async_copy(pref.at[idx], bbuf.at[t, :, pl.ds(0, 1), :], gsems[t])
      pltpu.async_copy(ctxt.at[idx], bbuf.at[t, :, pl.ds(1, N_CTX), :], gsems[t])
      pltpu.async_copy(suft.at[idx], bbuf.at[t, :, pl.ds(1 + N_CTX, SUF), :], gsems[t])

    def wait_gather(t):
      pltpu.make_async_copy(
          pref.at[pl.ds(0, C)], bbuf.at[t, :, pl.ds(0, 1), :], gsems[t]).wait()
      pltpu.make_async_copy(
          ctxt.at[pl.ds(0, C)], bbuf.at[t, :, pl.ds(1, N_CTX), :], gsems[t]).wait()
      pltpu.make_async_copy(
          suft.at[pl.ds(0, C)], bbuf.at[t, :, pl.ds(1 + N_CTX, SUF), :], gsems[t]).wait()

    def start_write(g, t):
      i0 = i0_tile + g * C
      return pltpu.async_copy(
          bbuf.at[t, :, pl.ds(0, SEQ), :], out.at[pl.ds(i0, C)], wsems[t])

    def loop_body(g, carry):
      start_gather(g, 0)
      wait_gather(0)
      start_write(g, 0).wait()
      return carry

    lax.fori_loop(0, NCHUNK, loop_body, 0)

  @pl.when(v == 0)
  def _():
    run(pn, cn, sn)

  @pl.when(v == 1)
  def _():
    run(pp, cp, sp)

  @pl.when(v == 2)
  def _():
    run(pe, ce, se)

  @pl.when(v == 3)
  def _():
    run(ps, cs, ss)


_gather_call = functools.partial(
    pl.kernel,
    mesh=plsc.VectorSubcoreMesh(core_axis_name="c", subcore_axis_name="s"),
    out_type=jax.ShapeDtypeStruct((NV * B, SEQ, D), jnp.float32),
    scratch_types=[
        pltpu.VMEM((IDX_ROWS, C), jnp.int32),
        pltpu.VMEM((1, C, 80, D), jnp.float32),
        pltpu.SemaphoreType.DMA,
        pltpu.SemaphoreType.DMA,
        pltpu.SemaphoreType.DMA,
        pltpu.SemaphoreType.DMA,
    ],
)(_sc_body)


@jax.jit
def kernel(cls_id, ctx_pos, ctx_neg, ctx_evi, ctx_sub,
           prefix_pos, suffix_pos, prefix_neg, suffix_neg,
           prefix_evi, suffix_evi, prefix_sub, suffix_sub):
  cls2d = cls_id.astype(jnp.int32).reshape(B // C, C)
  return _gather_call(
      cls2d,
      prefix_neg, ctx_neg, suffix_neg,
      prefix_pos, ctx_pos, suffix_pos,
      prefix_evi, ctx_evi, suffix_evi,
      prefix_sub, ctx_sub, suffix_sub,
  )


# speedup vs baseline: 1.0071x; 1.0071x over previous
import functools

import jax
import jax.numpy as jnp
from jax import lax
from jax.experimental import pallas as pl
from jax.experimental.pallas import tpu as pltpu
from jax.experimental.pallas import tpu_sc as plsc

N_CLS = 10000
N_CTX = 16
SUF = 60
SEQ = 77
D = 128
B = 1024
NV = 4

NW = 32
PER_TILE = NV * B // NW
NBUF = 8
NROUND = PER_TILE // NBUF
TILES_PER_V = NW // NV


def _sc_body(cls_ids, pn, cn, sn, pp, cp, sp, pe, ce, se, ps, cs, ss,
             out, idx_v, bbuf, gsem, wsem):
  cid = lax.axis_index("c")
  sid = lax.axis_index("s")
  wid = sid * 2 + cid
  v = wid // TILES_PER_V
  b0 = (wid % TILES_PER_V) * PER_TILE
  i0_tile = wid * PER_TILE

  pltpu.sync_copy(cls_ids.at[pl.ds(b0, PER_TILE)], idx_v.at[pl.ds(0, PER_TILE)])

  def run(pref, ctxt, suft):
    def start_gathers(iv, i, t):
      pltpu.async_copy(pref.at[iv], bbuf.at[t, pl.ds(0, 1), :], gsem.at[t])
      pltpu.async_copy(ctxt.at[iv], bbuf.at[t, pl.ds(1, N_CTX), :], gsem.at[t])
      pltpu.async_copy(suft.at[iv], bbuf.at[t, pl.ds(1 + N_CTX, SUF), :], gsem.at[t])

    def wait_gathers(t):
      pltpu.make_async_copy(
          pref.at[0], bbuf.at[t, pl.ds(0, 1), :], gsem.at[t]).wait()
      pltpu.make_async_copy(
          ctxt.at[0], bbuf.at[t, pl.ds(1, N_CTX), :], gsem.at[t]).wait()
      pltpu.make_async_copy(
          suft.at[0], bbuf.at[t, pl.ds(1 + N_CTX, SUF), :], gsem.at[t]).wait()

    def start_write(i, t):
      pltpu.async_copy(
          bbuf.at[t, pl.ds(0, SEQ), :], out.at[i0_tile + i], wsem.at[t])

    def wait_write(i, t):
      pltpu.make_async_copy(
          bbuf.at[t, pl.ds(0, SEQ), :], out.at[i0_tile + i], wsem.at[t]).wait()

    ivec0 = idx_v[pl.ds(0, 16)]
    for t in range(NBUF):
      start_gathers(ivec0[t], t, t)

    def loop_body(r, carry):
      for t in range(NBUF):
        wait_gathers(t)
        start_write(r * NBUF + t, t)

      @pl.when(r + 1 < NROUND)
      def _():
        base = (r + 1) * NBUF
        ivec = idx_v[pl.ds(base, 16)]
        for t in range(NBUF):
          wait_write(r * NBUF + t, t)
          start_gathers(ivec[t], base + t, t)

      return carry

    lax.fori_loop(0, NROUND, loop_body, 0)

    for t in range(NBUF):
      wait_write((NROUND - 1) * NBUF + t, t)

  @pl.when(v == 0)
  def _():
    run(pn, cn, sn)

  @pl.when(v == 1)
  def _():
    run(pp, cp, sp)

  @pl.when(v == 2)
  def _():
    run(pe, ce, se)

  @pl.when(v == 3)
  def _():
    run(ps, cs, ss)


_gather_call = functools.partial(
    pl.kernel,
    mesh=plsc.VectorSubcoreMesh(core_axis_name="c", subcore_axis_name="s"),
    out_type=jax.ShapeDtypeStruct((NV * B, SEQ, D), jnp.float32),
    scratch_types=[
        pltpu.VMEM((PER_TILE + 16,), jnp.int32),
        pltpu.VMEM((NBUF, 80, D), jnp.float32),
        pltpu.SemaphoreType.DMA((NBUF,)),
        pltpu.SemaphoreType.DMA((NBUF,)),
    ],
)(_sc_body)


@jax.jit
def kernel(cls_id, ctx_pos, ctx_neg, ctx_evi, ctx_sub,
           prefix_pos, suffix_pos, prefix_neg, suffix_neg,
           prefix_evi, suffix_evi, prefix_sub, suffix_sub):
  cls32 = cls_id.astype(jnp.int32)
  return _gather_call(
      cls32,
      prefix_neg, ctx_neg, suffix_neg,
      prefix_pos, ctx_pos, suffix_pos,
      prefix_evi, ctx_evi, suffix_evi,
      prefix_sub, ctx_sub, suffix_sub,
  )


# speedup vs baseline: 1.0166x; 1.0094x over previous
import functools

import jax
import jax.numpy as jnp
from jax import lax
from jax.experimental import pallas as pl
from jax.experimental.pallas import tpu as pltpu
from jax.experimental.pallas import tpu_sc as plsc

N_CLS = 10000
N_CTX = 16
SUF = 60
SEQ = 77
D = 128
B = 1024
NV = 4

NW = 32
PER_TILE = NV * B // NW
NBUF = 8
NROUND = PER_TILE // NBUF
TILES_PER_V = NW // NV


def _sc_body(cls_ids, pn, cn, sn, pp, cp, sp, pe, ce, se, ps, cs, ss,
             out, idx_v, sbuf, gsem, wsem):
  cid = lax.axis_index("c")
  sid = lax.axis_index("s")
  wid = sid * 2 + cid
  bbuf = sbuf.at[sid]
  v = wid // TILES_PER_V
  b0 = (wid % TILES_PER_V) * PER_TILE
  i0_tile = wid * PER_TILE

  pltpu.sync_copy(cls_ids.at[pl.ds(b0, PER_TILE)], idx_v.at[pl.ds(0, PER_TILE)])

  def run(pref, ctxt, suft):
    def start_gathers(iv, i, t):
      pltpu.async_copy(pref.at[iv], bbuf.at[t, pl.ds(0, 1), :], gsem.at[t])
      pltpu.async_copy(ctxt.at[iv], bbuf.at[t, pl.ds(1, N_CTX), :], gsem.at[t])
      pltpu.async_copy(suft.at[iv], bbuf.at[t, pl.ds(1 + N_CTX, SUF), :], gsem.at[t])

    def wait_gathers(t):
      pltpu.make_async_copy(
          pref.at[0], bbuf.at[t, pl.ds(0, 1), :], gsem.at[t]).wait()
      pltpu.make_async_copy(
          ctxt.at[0], bbuf.at[t, pl.ds(1, N_CTX), :], gsem.at[t]).wait()
      pltpu.make_async_copy(
          suft.at[0], bbuf.at[t, pl.ds(1 + N_CTX, SUF), :], gsem.at[t]).wait()

    def start_write(i, t):
      pltpu.async_copy(
          bbuf.at[t, pl.ds(0, SEQ), :], out.at[i0_tile + i], wsem.at[t])

    def wait_write(i, t):
      pltpu.make_async_copy(
          bbuf.at[t, pl.ds(0, SEQ), :], out.at[i0_tile + i], wsem.at[t]).wait()

    ivec0 = idx_v[pl.ds(0, 16)]
    for t in range(NBUF):
      start_gathers(ivec0[t], t, t)

    def loop_body(r, carry):
      for t in range(NBUF):
        wait_gathers(t)
        start_write(r * NBUF + t, t)

      @pl.when(r + 1 < NROUND)
      def _():
        base = (r + 1) * NBUF
        ivec = idx_v[pl.ds(base, 16)]
        for t in range(NBUF):
          wait_write(r * NBUF + t, t)
          start_gathers(ivec[t], base + t, t)

      return carry

    lax.fori_loop(0, NROUND, loop_body, 0)

    for t in range(NBUF):
      wait_write((NROUND - 1) * NBUF + t, t)

  @pl.when(v == 0)
  def _():
    run(pn, cn, sn)

  @pl.when(v == 1)
  def _():
    run(pp, cp, sp)

  @pl.when(v == 2)
  def _():
    run(pe, ce, se)

  @pl.when(v == 3)
  def _():
    run(ps, cs, ss)


_gather_call = functools.partial(
    pl.kernel,
    mesh=plsc.VectorSubcoreMesh(core_axis_name="c", subcore_axis_name="s"),
    out_type=jax.ShapeDtypeStruct((NV * B, SEQ, D), jnp.float32),
    scratch_types=[
        pltpu.VMEM((PER_TILE + 16,), jnp.int32),
        pltpu.VMEM_SHARED((16, NBUF, 80, D), jnp.float32),
        pltpu.SemaphoreType.DMA((NBUF,)),
        pltpu.SemaphoreType.DMA((NBUF,)),
    ],
)(_sc_body)


@jax.jit
def kernel(cls_id, ctx_pos, ctx_neg, ctx_evi, ctx_sub,
           prefix_pos, suffix_pos, prefix_neg, suffix_neg,
           prefix_evi, suffix_evi, prefix_sub, suffix_sub):
  cls32 = cls_id.astype(jnp.int32)
  return _gather_call(
      cls32,
      prefix_neg, ctx_neg, suffix_neg,
      prefix_pos, ctx_pos, suffix_pos,
      prefix_evi, ctx_evi, suffix_evi,
      prefix_sub, ctx_sub, suffix_sub,
  )
